# baseline (speedup 1.0000x reference)
import jax
import jax.numpy as jnp
from jax import lax
from jax.experimental import pallas as pl
from jax.experimental.pallas import tpu as pltpu
from jax.experimental.pallas import tpu_sc as plsc

_B = 4096
_DIM = 1024
_POOL = 100000

_BT = 4096
_PB = 512
_NPB = -(-_POOL // _PB)


def _xnorm_body(x_ref, o_ref):
    x = x_ref[...]
    xn = x / (jnp.sqrt(jnp.sum(x * x, axis=1, keepdims=True)) + 1e-12)
    o_ref[...] = xn.astype(jnp.bfloat16)


def _xnorm_call(x):
    return pl.pallas_call(
        _xnorm_body,
        grid=(_B // _BT,),
        in_specs=[pl.BlockSpec((_BT, _DIM), lambda i: (i, 0))],
        out_specs=pl.BlockSpec((_BT, _DIM), lambda i: (i, 0)),
        out_shape=jax.ShapeDtypeStruct((_B, _DIM), jnp.bfloat16),
    )(x)


def _simargmax_body(xnb_ref, p_ref, idx_ref, mv_ref, ai_ref, s0_ref, s1_ref):
    j = pl.program_id(0)

    @pl.when(j == 0)
    def _():
        mv_ref[...] = jnp.full((_B, 1), -jnp.inf, jnp.float32)
        ai_ref[...] = jnp.zeros((_B, 1), jnp.int32)

    def _dot_into(sref):
        p = p_ref[...]
        pn = p / (jnp.sqrt(jnp.sum(p * p, axis=1, keepdims=True)) + 1e-12)
        sref[...] = lax.dot_general(xnb_ref[...], pn.astype(jnp.bfloat16),
                                    (((1,), (1,)), ((), ())),
                                    preferred_element_type=jnp.float32)

    @pl.when((j < _NPB) & (j % 2 == 0))
    def _():
        _dot_into(s0_ref)

    @pl.when((j < _NPB) & (j % 2 == 1))
    def _():
        _dot_into(s1_ref)

    lcols = lax.broadcasted_iota(jnp.int32, (_B, _PB), 1)

    def _update(sv, jj):
        m = jnp.max(sv, axis=1, keepdims=True)
        amax = jnp.min(jnp.where(sv == m, lcols, jnp.int32(_PB)),
                       axis=1, keepdims=True) + jj * _PB
        better = m > mv_ref[...]
        ai_ref[...] = jnp.where(better, amax, ai_ref[...])
        mv_ref[...] = jnp.where(better, m, mv_ref[...])

    @pl.when((j > 0) & (j < _NPB) & (j % 2 == 1))
    def _():
        _update(s0_ref[...], j - 1)

    @pl.when((j > 0) & (j < _NPB) & (j % 2 == 0))
    def _():
        _update(s1_ref[...], j - 1)

    @pl.when(j == _NPB)
    def _():
        last_ref = s1_ref if (_NPB - 1) % 2 else s0_ref
        sv = jnp.where(lcols + (_NPB - 1) * _PB < _POOL, last_ref[...],
                       -jnp.inf)
        _update(sv, _NPB - 1)
        idx_ref[...] = ai_ref[...]


def _argmax_call(x, pool):
    xnb = _xnorm_call(x)
    return pl.pallas_call(
        _simargmax_body,
        grid=(_NPB + 1,),
        in_specs=[pl.BlockSpec((_B, _DIM), lambda j: (0, 0)),
                  pl.BlockSpec((_PB, _DIM),
                               lambda j: (jnp.minimum(j, _NPB - 1), 0))],
        out_specs=pl.BlockSpec((_B, 1), lambda j: (0, 0)),
        out_shape=jax.ShapeDtypeStruct((_B, 1), jnp.int32),
        scratch_shapes=[pltpu.VMEM((_B, 1), jnp.float32),
                        pltpu.VMEM((_B, 1), jnp.int32),
                        pltpu.VMEM((_B, _PB), jnp.float32),
                        pltpu.VMEM((_B, _PB), jnp.float32)],
        compiler_params=pltpu.CompilerParams(
            dimension_semantics=("arbitrary",),
            vmem_limit_bytes=100 * 1024 * 1024),
    )(xnb, pool)


_NC = 2
_NS = 16
_NW = _NC * _NS
_BPW = _B // _NW
_CH = 64
_NCH = _BPW // _CH


def _gather_body(pool_hbm, idx_hbm, out_hbm, idx_v, rows_v, sem):
    wid = lax.axis_index("s") * _NC + lax.axis_index("c")
    pltpu.sync_copy(idx_hbm.at[wid], idx_v)
    for c in range(_NCH):
        pltpu.async_copy(pool_hbm.at[idx_v.at[c]], rows_v, sem).wait()
        pltpu.sync_copy(rows_v, out_hbm.at[pl.ds(wid * _BPW + c * _CH, _CH)])


def _gather_call(pool, idx):
    mesh = plsc.VectorSubcoreMesh(core_axis_name="c", subcore_axis_name="s")
    kfn = pl.kernel(
        _gather_body,
        mesh=mesh,
        out_type=jax.ShapeDtypeStruct((_B, _DIM), jnp.float32),
        scratch_types=[pltpu.VMEM((_NCH, _CH), jnp.int32),
                       pltpu.VMEM((_CH, _DIM), jnp.float32),
                       pltpu.SemaphoreType.DMA],
    )
    return kfn(pool, idx.reshape(_NW, _NCH, _CH))


def _avg_body(x_ref, g_ref, o_ref):
    o_ref[...] = (x_ref[...] + g_ref[...]) * 0.5


def _avg_call(x, g):
    return pl.pallas_call(
        _avg_body,
        grid=(_B // _BT,),
        in_specs=[pl.BlockSpec((_BT, _DIM), lambda i: (i, 0)),
                  pl.BlockSpec((_BT, _DIM), lambda i: (i, 0))],
        out_specs=pl.BlockSpec((_BT, _DIM), lambda i: (i, 0)),
        out_shape=jax.ShapeDtypeStruct((_B, _DIM), jnp.float32),
    )(x, g)


def kernel(x, pool):
    idx = _argmax_call(x, pool)
    g = _gather_call(pool, idx)
    return _avg_call(x, g)


# speedup vs baseline: 1.0990x; 1.0990x over previous
import jax
import jax.numpy as jnp
from jax import lax
from jax.experimental import pallas as pl
from jax.experimental.pallas import tpu as pltpu
from jax.experimental.pallas import tpu_sc as plsc

_B = 4096
_DIM = 1024
_POOL = 100000

_BT = 4096
_PB = 1024
_NPB = -(-_POOL // _PB)


def _xnorm_body(x_ref, o_ref):
    x = x_ref[...]
    xn = x / (jnp.sqrt(jnp.sum(x * x, axis=1, keepdims=True)) + 1e-12)
    o_ref[...] = xn.astype(jnp.bfloat16)


def _xnorm_call(x):
    return pl.pallas_call(
        _xnorm_body,
        grid=(_B // _BT,),
        in_specs=[pl.BlockSpec((_BT, _DIM), lambda i: (i, 0))],
        out_specs=pl.BlockSpec((_BT, _DIM), lambda i: (i, 0)),
        out_shape=jax.ShapeDtypeStruct((_B, _DIM), jnp.bfloat16),
    )(x)


def _simargmax_body(xnb_ref, p_ref, idx_ref, mv_ref, ai_ref):
    j = pl.program_id(0)

    @pl.when(j == 0)
    def _():
        mv_ref[...] = jnp.full((_B, 1), -jnp.inf, jnp.float32)
        ai_ref[...] = jnp.zeros((_B, 1), jnp.int32)

    p = p_ref[...]
    pn = p / (jnp.sqrt(jnp.sum(p * p, axis=1, keepdims=True)) + 1e-12)
    s = lax.dot_general(xnb_ref[...], pn.astype(jnp.bfloat16),
                        (((1,), (1,)), ((), ())),
                        preferred_element_type=jnp.float32)
    lcols = lax.broadcasted_iota(jnp.int32, (_B, _PB), 1)

    def _update(sv):
        m = jnp.max(sv, axis=1, keepdims=True)
        amax = jnp.min(jnp.where(sv == m, lcols, jnp.int32(_PB)),
                       axis=1, keepdims=True) + j * _PB
        better = m > mv_ref[...]
        ai_ref[...] = jnp.where(better, amax, ai_ref[...])
        mv_ref[...] = jnp.where(better, m, mv_ref[...])

    @pl.when(j < _NPB - 1)
    def _():
        _update(s)

    @pl.when(j == _NPB - 1)
    def _():
        _update(jnp.where(lcols + j * _PB < _POOL, s, -jnp.inf))
        idx_ref[...] = ai_ref[...]


def _argmax_call(x, pool):
    xnb = _xnorm_call(x)
    return pl.pallas_call(
        _simargmax_body,
        grid=(_NPB,),
        in_specs=[pl.BlockSpec((_B, _DIM), lambda j: (0, 0)),
                  pl.BlockSpec((_PB, _DIM), lambda j: (j, 0))],
        out_specs=pl.BlockSpec((_B, 1), lambda j: (0, 0)),
        out_shape=jax.ShapeDtypeStruct((_B, 1), jnp.int32),
        scratch_shapes=[pltpu.VMEM((_B, 1), jnp.float32),
                        pltpu.VMEM((_B, 1), jnp.int32)],
        compiler_params=pltpu.CompilerParams(
            dimension_semantics=("arbitrary",),
            vmem_limit_bytes=100 * 1024 * 1024),
    )(xnb, pool)


_NC = 2
_NS = 16
_NW = _NC * _NS
_BPW = _B // _NW
_CH = 64
_NCH = _BPW // _CH


def _gather_body(pool_hbm, idx_hbm, out_hbm, idx_v, rows_v, sem):
    wid = lax.axis_index("s") * _NC + lax.axis_index("c")
    pltpu.sync_copy(idx_hbm.at[wid], idx_v)
    for c in range(_NCH):
        pltpu.async_copy(pool_hbm.at[idx_v.at[c]], rows_v, sem).wait()
        pltpu.sync_copy(rows_v, out_hbm.at[pl.ds(wid * _BPW + c * _CH, _CH)])


def _gather_call(pool, idx):
    mesh = plsc.VectorSubcoreMesh(core_axis_name="c", subcore_axis_name="s")
    kfn = pl.kernel(
        _gather_body,
        mesh=mesh,
        out_type=jax.ShapeDtypeStruct((_B, _DIM), jnp.float32),
        scratch_types=[pltpu.VMEM((_NCH, _CH), jnp.int32),
                       pltpu.VMEM((_CH, _DIM), jnp.float32),
                       pltpu.SemaphoreType.DMA],
    )
    return kfn(pool, idx.reshape(_NW, _NCH, _CH))


def _avg_body(x_ref, g_ref, o_ref):
    o_ref[...] = (x_ref[...] + g_ref[...]) * 0.5


def _avg_call(x, g):
    return pl.pallas_call(
        _avg_body,
        grid=(_B // _BT,),
        in_specs=[pl.BlockSpec((_BT, _DIM), lambda i: (i, 0)),
                  pl.BlockSpec((_BT, _DIM), lambda i: (i, 0))],
        out_specs=pl.BlockSpec((_BT, _DIM), lambda i: (i, 0)),
        out_shape=jax.ShapeDtypeStruct((_B, _DIM), jnp.float32),
    )(x, g)


def kernel(x, pool):
    idx = _argmax_call(x, pool)
    g = _gather_call(pool, idx)
    return _avg_call(x, g)
